# Initial kernel scaffold; baseline (speedup 1.0000x reference)
import jax
import jax.numpy as jnp
from jax.experimental import pallas as pl


def kernel(inputs, token_table, pos_table):
    raise NotImplementedError("write your pallas kernel here")


# speedup vs baseline: 1.4280x; 1.4280x over previous
import functools

import jax
import jax.numpy as jnp
from jax import lax
from jax.experimental import pallas as pl
from jax.experimental.pallas import tpu as pltpu
from jax.experimental.pallas import tpu_sc as plsc

VOCAB = 1000000
SEQ_LEN = 200
EMBED_DIM = 32
BATCH = 4096

NUM_CORES = 2
NUM_SUBCORES = 16
NUM_WORKERS = NUM_CORES * NUM_SUBCORES

SEQ_PER_WORKER = BATCH // NUM_WORKERS
CHUNK_SEQS = 8
ROWS_PER_CHUNK = CHUNK_SEQS * SEQ_LEN
CHUNKS_PER_WORKER = SEQ_PER_WORKER // CHUNK_SEQS
NUM_CHUNKS = NUM_WORKERS * CHUNKS_PER_WORKER
GATHER_W = 64
GATHERS_PER_CHUNK = ROWS_PER_CHUNK // GATHER_W
LANES = 16


def _body(idx_hbm, tok_hbm, pos_hbm, out_hbm, idx_v, rows_v, pos_v, sem):
    wid = lax.axis_index("s") * NUM_CORES + lax.axis_index("c")
    pltpu.sync_copy(pos_hbm, pos_v)

    def chunk_body(c, _):
        chunk_id = wid * CHUNKS_PER_WORKER + c
        row0 = pl.multiple_of(chunk_id * ROWS_PER_CHUNK, ROWS_PER_CHUNK)
        pltpu.sync_copy(idx_hbm.at[chunk_id], idx_v)
        copies = []
        for j in range(GATHERS_PER_CHUNK):
            copies.append(
                pltpu.async_copy(
                    tok_hbm.at[idx_v.at[j]],
                    rows_v.at[pl.ds(j * GATHER_W, GATHER_W)],
                    sem,
                )
            )
        for cp in copies:
            cp.wait()

        def pos_body(p, carry):
            p0 = pos_v[p, pl.ds(0, LANES)]
            p1 = pos_v[p, pl.ds(LANES, LANES)]
            for s in range(CHUNK_SEQS):
                r = s * SEQ_LEN + p
                rows_v[r, pl.ds(0, LANES)] += p0
                rows_v[r, pl.ds(LANES, LANES)] += p1
            return carry

        lax.fori_loop(0, SEQ_LEN, pos_body, 0)
        pltpu.sync_copy(rows_v, out_hbm.at[pl.ds(row0, ROWS_PER_CHUNK)])
        return _

    lax.fori_loop(0, CHUNKS_PER_WORKER, chunk_body, 0)


_mesh = plsc.VectorSubcoreMesh(core_axis_name="c", subcore_axis_name="s")

_sc_call = functools.partial(
    pl.kernel,
    out_type=jax.ShapeDtypeStruct((BATCH * SEQ_LEN, EMBED_DIM), jnp.float32),
    mesh=_mesh,
    scratch_types=[
        pltpu.VMEM((GATHERS_PER_CHUNK, GATHER_W), jnp.int32),
        pltpu.VMEM((ROWS_PER_CHUNK, EMBED_DIM), jnp.float32),
        pltpu.VMEM((SEQ_LEN, EMBED_DIM), jnp.float32),
        pltpu.SemaphoreType.DMA,
    ],
    compiler_params=pltpu.CompilerParams(use_tc_tiling_on_sc=False),
)


@jax.jit
def kernel(inputs, token_table, pos_table):
    idx = inputs.astype(jnp.int32).reshape(
        NUM_CHUNKS, GATHERS_PER_CHUNK, GATHER_W
    )
    out_flat = _sc_call(_body)(idx, token_table, pos_table)
    return out_flat.reshape(BATCH, SEQ_LEN, EMBED_DIM)
